# baseline (speedup 1.0000x reference)
import functools

import jax
import jax.numpy as jnp
from jax import lax
from jax.experimental import pallas as pl
from jax.experimental.pallas import tpu as pltpu
from jax.experimental.pallas import tpu_sc as plsc

VOCAB = 1000000
EMBED_DIM = 32
BATCH = 16384
N_FIELDS = 26
PACK = 128 // EMBED_DIM

NW = 32
VBLK = 512
PSZ = VBLK * EMBED_DIM
NPIPE = 60
NEXTRA = VOCAB // VBLK - NPIPE * NW
V_TAIL = VOCAB - (VOCAB // VBLK) * VBLK
P_LEN = VOCAB * EMBED_DIM

BBLK = 256
B_FLAT = BATCH * N_FIELDS
N_UNITS = B_FLAT // BBLK
UPW = N_UNITS // NW
OSZ = BBLK * EMBED_DIM
OUT_LEN = B_FLAT * EMBED_DIM

_GATHER_DN = lax.GatherDimensionNumbers(
    offset_dims=(), collapsed_slice_dims=(0,), start_index_map=(0,))


def _lane_perm(v, perm2d):
    return lax.gather(v, perm2d, _GATHER_DN, (1,),
                      mode=lax.GatherScatterMode.PROMISE_IN_BOUNDS)


def _xpose16(rows):
    lanes = lax.iota(jnp.int32, 16)
    for d in (8, 4, 2, 1):
        mask = (lanes & d) == 0
        perm2d = (lanes ^ d)[:, None]
        nxt = list(rows)
        for r in range(16):
            if r & d:
                continue
            a, b = rows[r], rows[r + d]
            nxt[r] = jnp.where(mask, a, _lane_perm(b, perm2d))
            nxt[r + d] = jnp.where(mask, _lane_perm(a, perm2d), b)
        rows = nxt
    return rows


def _phase1(embT, wT, tail_p, p_out, eb, wb, pb, tb,
            sl0, sl1, so0, so1):
    wid = lax.axis_index("s") * 2 + lax.axis_index("c")
    sl = (sl0, sl1)
    so = (so0, so1)

    def issue_loads(panel, par):
        v0 = panel * VBLK
        pltpu.async_copy(embT.at[:, pl.ds(v0, VBLK)], eb.at[par], sl[par])
        pltpu.async_copy(wT.at[:, pl.ds(v0, VBLK)], wb.at[par], sl[par])

    def wait_loads(panel, par):
        v0 = panel * VBLK
        pltpu.make_async_copy(embT.at[:, pl.ds(v0, VBLK)], eb.at[par], sl[par]).wait()
        pltpu.make_async_copy(wT.at[:, pl.ds(v0, VBLK)], wb.at[par], sl[par]).wait()

    def compute_panel(par, nrows=VBLK // 16):
        def vb_body(j, c2):
            for ch in range(2):
                s16 = pl.ds(j * 16, 16)
                rows = [eb[par, ch * 16 + i, s16] * wb[par, ch * 16 + i, s16]
                        for i in range(16)]
                t = _xpose16(rows)
                for i in range(16):
                    pb[par, pl.ds((j * 16 + i) * EMBED_DIM + ch * 16, 16)] = t[i]
            return c2

        lax.fori_loop(0, nrows, vb_body, 0, unroll=4)

    for par in range(2):
        issue_loads(wid + par * NW, par)

    def blk_body(k2, carry):
        for par in range(2):
            k = k2 * 2 + par
            panel = wid + k * NW
            wait_loads(panel, par)

            @pl.when(k2 >= 1)
            def _():
                pltpu.make_async_copy(
                    pb.at[par], p_out.at[pl.ds(0, PSZ)], so[par]).wait()

            compute_panel(par)
            pltpu.async_copy(
                pb.at[par], p_out.at[pl.ds(panel * PSZ, PSZ)], so[par])

            @pl.when(k2 < NPIPE // 2 - 1)
            def _():
                issue_loads(wid + (k + 2) * NW, par)

        return carry

    lax.fori_loop(0, NPIPE // 2, blk_body, 0)
    for par in range(2):
        pltpu.make_async_copy(pb.at[par], p_out.at[pl.ds(0, PSZ)], so[par]).wait()

    extra = NPIPE * NW + wid
    pltpu.sync_copy(embT.at[:, pl.ds(extra * VBLK, VBLK)], eb.at[0])
    pltpu.sync_copy(wT.at[:, pl.ds(extra * VBLK, VBLK)], wb.at[0])
    compute_panel(0)
    pltpu.sync_copy(pb.at[0], p_out.at[pl.ds(extra * PSZ, PSZ)])

    @pl.when(wid == NW - 1)
    def _():
        last = NPIPE * NW + NW
        pltpu.sync_copy(embT.at[:, pl.ds(last * VBLK, VBLK)], eb.at[1])
        pltpu.sync_copy(wT.at[:, pl.ds(last * VBLK, VBLK)], wb.at[1])
        compute_panel(1)
        pltpu.sync_copy(pb.at[1], p_out.at[pl.ds(last * PSZ, PSZ)])

        pltpu.sync_copy(tail_p, tb)
        for r in range(V_TAIL * EMBED_DIM // 128):
            for h in range(8):
                pb[0, pl.ds(r * 128 + h * 16, 16)] = tb[r, pl.ds(h * 16, 16)]
        pltpu.sync_copy(
            pb.at[0, pl.ds(0, V_TAIL * EMBED_DIM)],
            p_out.at[pl.ds((last + 1) * PSZ, V_TAIL * EMBED_DIM)])


def _phase2(idxT, p_in, out, ib, tid, offb, gb, ob,
            sg0, sg1, so0, so1):
    wid = lax.axis_index("s") * 2 + lax.axis_index("c")
    sg = (sg0, sg1)
    so = (so0, so1)
    bc_per_f = BATCH // BBLK

    def unit_fb(k):
        u = wid + k * NW
        return u // bc_per_f, lax.rem(u, bc_per_f) * BBLK

    def prep(k, par):
        f, b0 = unit_fb(k)
        pltpu.sync_copy(idxT.at[pl.ds(f, 1), pl.ds(b0, BBLK)], ib.at[par])
        for h in range(BBLK // 16):
            s16 = pl.ds(h * 16, 16)
            iv = ib[par, 0, s16]
            tid[par, h // 8, pl.ds((h % 8) * 16, 16)] = (
                lax.shift_right_logical(iv, 2))
            offb[par, s16] = lax.rem(iv, PACK) * EMBED_DIM
        for j in range(BBLK // 128):
            pltpu.async_copy(
                p_in.at[tid.at[par, j]],
                gb.at[par, pl.ds(j * 128, 128)], sg[par])

    def wait_gathers(par):
        for j in range(BBLK // 128):
            pltpu.make_async_copy(
                p_in.at[tid.at[par, j]],
                gb.at[par, pl.ds(j * 128, 128)], sg[par]).wait()

    for par in range(2):
        prep(par, par)

    def unit_body(k2, carry):
        for par in range(2):
            k = k2 * 2 + par
            f, b0 = unit_fb(k)
            wait_gathers(par)

            @pl.when(k2 >= 1)
            def _():
                pltpu.make_async_copy(
                    ob.at[par], out.at[0, :, pl.ds(0, BBLK)], so[par]).wait()

            def g_body(g, c2):
                offv = offb[par, pl.ds(g * 16, 16)]
                offs = [offv[kk] for kk in range(16)]
                for ch in range(2):
                    rows = [
                        gb[par, g * 16 + kk, pl.ds(offs[kk] + ch * 16, 16)]
                        for kk in range(16)
                    ]
                    t = _xpose16(rows)
                    for i in range(16):
                        ob[par, ch * 16 + i, pl.ds(g * 16, 16)] = t[i]
                return c2

            lax.fori_loop(0, BBLK // 16, g_body, 0, unroll=2)
            pltpu.async_copy(
                ob.at[par], out.at[f, :, pl.ds(b0, BBLK)], so[par])

            @pl.when(k2 < UPW // 2 - 1)
            def _():
                prep(k + 2, par)

        return carry

    lax.fori_loop(0, UPW // 2, unit_body, 0)
    for par in range(2):
        pltpu.make_async_copy(
            ob.at[par], out.at[0, :, pl.ds(0, BBLK)], so[par]).wait()


_MESH = plsc.VectorSubcoreMesh(core_axis_name="c", subcore_axis_name="s")

_p1 = functools.partial(
    pl.kernel,
    out_type=jax.ShapeDtypeStruct((P_LEN,), jnp.float32),
    mesh=_MESH,
    scratch_types=[
        pltpu.VMEM((2, EMBED_DIM, VBLK), jnp.float32),
        pltpu.VMEM((2, EMBED_DIM, VBLK), jnp.float32),
        pltpu.VMEM((2, PSZ), jnp.float32),
        pltpu.VMEM((V_TAIL * EMBED_DIM // 128, 128), jnp.float32),
        pltpu.SemaphoreType.DMA,
        pltpu.SemaphoreType.DMA,
        pltpu.SemaphoreType.DMA,
        pltpu.SemaphoreType.DMA,
    ],
)(_phase1)

_p2 = functools.partial(
    pl.kernel,
    out_type=jax.ShapeDtypeStruct((N_FIELDS, EMBED_DIM, BATCH), jnp.float32),
    mesh=_MESH,
    scratch_types=[
        pltpu.VMEM((2, 1, BBLK), jnp.int32),
        pltpu.VMEM((2, BBLK // 128, 128), jnp.int32),
        pltpu.VMEM((2, BBLK), jnp.int32),
        pltpu.VMEM((2, BBLK, 128), jnp.float32),
        pltpu.VMEM((2, EMBED_DIM, BBLK), jnp.float32),
        pltpu.SemaphoreType.DMA,
        pltpu.SemaphoreType.DMA,
        pltpu.SemaphoreType.DMA,
        pltpu.SemaphoreType.DMA,
    ],
)(_phase2)


@jax.jit
def kernel(inputs, embeddings, w):
    embT = embeddings.T
    wT = w.T
    tail_v0 = (VOCAB // VBLK) * VBLK
    tail_p = (embeddings[tail_v0:] * w[tail_v0:]).reshape(
        V_TAIL * EMBED_DIM // 128, 128)
    idxT = inputs.astype(jnp.int32).T
    p = _p1(embT, wT, tail_p)
    p2d = p.reshape(P_LEN // 128, 128)
    outT = _p2(idxT, p2d)
    return outT.transpose(2, 0, 1)
